# baseline (speedup 1.0000x reference)
import functools

import jax
import jax.numpy as jnp
from jax import lax
from jax.experimental import pallas as pl
from jax.experimental.pallas import tpu as pltpu
from jax.experimental.pallas import tpu_sc as plsc

_N = 10000
_D = 128
_N_P = 10240
_E = 320000
_E_P = 327680
_LANES = 128
_CW = 64
_CROWS = _E_P // _CW
_NC = 2
_NS = 16
_TILES = _NC * _NS
_CROWS_PT = _CROWS // _TILES
_NROWS = _N_P // _LANES
_ACC_PT = _N_P // _NS
_TRASH = _N

_mesh = plsc.VectorSubcoreMesh(core_axis_name="c", subcore_axis_name="s")


@functools.partial(
    pl.kernel,
    out_type=jax.ShapeDtypeStruct((_TILES * _N_P,), jnp.float32),
    mesh=_mesh,
    compiler_params=pltpu.CompilerParams(needs_layout_passes=False),
    scratch_types=[
        pltpu.VMEM((_CROWS_PT, _CW), jnp.int32),
        pltpu.VMEM((_N_P,), jnp.float32),
        pltpu.SemaphoreType.DMA,
    ],
)
def _deg_kernel(dst_hbm, out_hbm, dst_v, cnt_v, sem):
    cid = lax.axis_index("c")
    sid = lax.axis_index("s")
    tid = cid * _NS + sid
    zero16 = jnp.zeros((16,), jnp.float32)

    def _zcnt(i, _):
        cnt_v[pl.ds(i * 16, 16)] = zero16
        return 0
    lax.fori_loop(0, _N_P // 16, _zcnt, 0)

    pltpu.sync_copy(dst_hbm.at[pl.ds(tid * _CROWS_PT, _CROWS_PT)], dst_v)

    ones16 = jnp.ones((16,), jnp.float32)

    def _cnt(i, _):
        r = i // 4
        k = i - r * 4
        idx = dst_v[r, pl.ds(k * 16, 16)]
        plsc.addupdate_scatter(cnt_v, [idx], ones16)
        return 0
    lax.fori_loop(0, _CROWS_PT * 4, _cnt, 0)

    pltpu.sync_copy(cnt_v, out_hbm.at[pl.ds(tid * _N_P, _N_P)])


def _tc_deg_reduce(degp):
    def body(p_ref, o_ref):
        o_ref[...] = jnp.sum(p_ref[...], axis=0)

    return pl.pallas_call(
        body,
        out_shape=jax.ShapeDtypeStruct((_NROWS, _LANES), jnp.float32),
    )(degp)


@functools.partial(
    pl.kernel,
    out_type=jax.ShapeDtypeStruct((_NC, _N_P, _D), jnp.float32),
    mesh=_mesh,
    compiler_params=pltpu.CompilerParams(needs_layout_passes=False),
    scratch_types=[
        pltpu.VMEM((_CROWS_PT // 4, _CW), jnp.int32),
        pltpu.VMEM((_CROWS_PT // 4, _CW), jnp.int32),
        pltpu.VMEM((_CW, _D), jnp.float32),
        pltpu.VMEM((_CW, _D), jnp.float32),
        pltpu.VMEM((_CW, _D), jnp.float32),
        pltpu.VMEM((_CW, _D), jnp.float32),
        pltpu.VMEM_SHARED((_N_P, _D), jnp.float32),
        pltpu.SemaphoreType.DMA,
        pltpu.SemaphoreType.DMA,
        pltpu.SemaphoreType.DMA,
        pltpu.SemaphoreType.DMA,
        pltpu.SemaphoreType.DMA,
        pltpu.SemaphoreType.DMA,
        pltpu.SemaphoreType.DMA,
        pltpu.SemaphoreType.DMA,
    ],
)
def _agg_kernel(g_hbm, src_hbm, dst_hbm, out_hbm,
                src_v, dst_v, buf0, buf1, buf2, buf3, acc_sh,
                gs0, gs1, gs2, gs3, ss0, ss1, ss2, ss3):
    cid = lax.axis_index("c")
    sid = lax.axis_index("s")
    tid = cid * _NS + sid
    bufs = [buf0, buf1, buf2, buf3]
    gs = [gs0, gs1, gs2, gs3]
    ss = [ss0, ss1, ss2, ss3]
    zero16 = jnp.zeros((16,), jnp.float32)

    def _z(i, _):
        r = i // 8
        k = i - r * 8
        buf0[r, pl.ds(k * 16, 16)] = zero16
        return 0
    lax.fori_loop(0, _CW * 8, _z, 0)

    def _zs(b, _):
        pltpu.sync_copy(buf0, acc_sh.at[pl.ds(sid * _ACC_PT + b * _CW, _CW)])
        return 0
    lax.fori_loop(0, _ACC_PT // _CW, _zs, 0)
    plsc.subcore_barrier()

    def _gather(c, b):
        pltpu.async_copy(g_hbm.at[src_v.at[c]], bufs[b], gs[b])

    def _gwait(b):
        pltpu.make_async_copy(g_hbm.at[src_v.at[0]], bufs[b], gs[b]).wait()

    def _scat(c, b):
        pltpu.async_copy(bufs[b], acc_sh.at[dst_v.at[c]], ss[b], add=True)

    def _swait(b):
        pltpu.make_async_copy(bufs[b], acc_sh.at[dst_v.at[0]], ss[b]).wait()

    nh = _CROWS_PT // 4
    nj = nh // 4
    for h in range(4):
        base = tid * _CROWS_PT + h * nh
        pltpu.sync_copy(src_hbm.at[pl.ds(base, nh)], src_v)
        pltpu.sync_copy(dst_hbm.at[pl.ds(base, nh)], dst_v)
        _gather(0, 0)
        _gather(1, 1)

        def _step(j, _):
            c0 = 4 * j
            _gwait(0)
            _scat(c0, 0)

            @pl.when(j > 0)
            def _():
                _swait(2)
            _gather(c0 + 2, 2)
            _gwait(1)
            _scat(c0 + 1, 1)

            @pl.when(j > 0)
            def _():
                _swait(3)
            _gather(c0 + 3, 3)
            _gwait(2)
            _scat(c0 + 2, 2)
            _swait(0)

            @pl.when(j < nj - 1)
            def _():
                _gather(c0 + 4, 0)
            _gwait(3)
            _scat(c0 + 3, 3)
            _swait(1)

            @pl.when(j < nj - 1)
            def _():
                _gather(c0 + 5, 1)
            return 0
        lax.fori_loop(0, nj, _step, 0)
        _swait(2)
        _swait(3)

    plsc.subcore_barrier()
    pltpu.sync_copy(acc_sh.at[pl.ds(sid * _ACC_PT, _ACC_PT)],
                    out_hbm.at[cid, pl.ds(sid * _ACC_PT, _ACC_PT)])


_BM = 1024
_BM3 = 1000


def _tc_layer1(x_pad, w1, deg):
    def body(x_ref, w_ref, d_ref, g_ref):
        dis = lax.rsqrt(d_ref[...] + 1.0)
        h = jnp.dot(x_ref[...], w_ref[...], preferred_element_type=jnp.float32)
        g_ref[...] = h * dis

    return pl.pallas_call(
        body,
        grid=(_N_P // _BM,),
        in_specs=[
            pl.BlockSpec((_BM, _D), lambda b: (b, 0)),
            pl.BlockSpec((_D, _D), lambda b: (0, 0)),
            pl.BlockSpec((_BM, 1), lambda b: (b, 0)),
        ],
        out_specs=pl.BlockSpec((_BM, _D), lambda b: (b, 0)),
        out_shape=jax.ShapeDtypeStruct((_N_P, _D), jnp.float32),
    )(x_pad, w1, deg)


def _tc_layer2(parts, g1, deg, w2, b1):
    def body(p_ref, g_ref, d_ref, w_ref, b_ref, o_ref):
        dis = lax.rsqrt(d_ref[...] + 1.0)
        s = p_ref[0] + p_ref[1] + g_ref[...]
        z = jnp.maximum(s * dis + b_ref[...], 0.0)
        o_ref[...] = jnp.dot(z, w_ref[...], preferred_element_type=jnp.float32) * dis

    return pl.pallas_call(
        body,
        grid=(_N_P // _BM,),
        in_specs=[
            pl.BlockSpec((_NC, _BM, _D), lambda b: (0, b, 0)),
            pl.BlockSpec((_BM, _D), lambda b: (b, 0)),
            pl.BlockSpec((_BM, 1), lambda b: (b, 0)),
            pl.BlockSpec((_D, _D), lambda b: (0, 0)),
            pl.BlockSpec((1, _D), lambda b: (0, 0)),
        ],
        out_specs=pl.BlockSpec((_BM, _D), lambda b: (b, 0)),
        out_shape=jax.ShapeDtypeStruct((_N_P, _D), jnp.float32),
    )(parts, g1, deg, w2, b1)


def _tc_layer3(parts, g2, deg, b2):
    def body(p_ref, g_ref, d_ref, b_ref, o_ref):
        dis = lax.rsqrt(d_ref[...] + 1.0)
        s = p_ref[0] + p_ref[1] + g_ref[...]
        o_ref[...] = s * dis + b_ref[...]

    return pl.pallas_call(
        body,
        grid=(_N // _BM3,),
        in_specs=[
            pl.BlockSpec((_NC, _BM3, _D), lambda b: (0, b, 0)),
            pl.BlockSpec((_BM3, _D), lambda b: (b, 0)),
            pl.BlockSpec((_BM3, 1), lambda b: (b, 0)),
            pl.BlockSpec((1, _D), lambda b: (0, 0)),
        ],
        out_specs=pl.BlockSpec((_BM3, _D), lambda b: (b, 0)),
        out_shape=jax.ShapeDtypeStruct((_N, _D), jnp.float32),
    )(parts, g2, deg, b2)


def kernel(x, edge_index, W1, b1, W2, b2):
    x = x.astype(jnp.float32)
    src = edge_index[0].astype(jnp.int32)
    dst = edge_index[1].astype(jnp.int32)
    src_p = jnp.concatenate(
        [src, jnp.zeros((_E_P - _E,), jnp.int32)]).reshape(_CROWS, _CW)
    dst_p = jnp.concatenate(
        [dst, jnp.full((_E_P - _E,), _TRASH, jnp.int32)]).reshape(_CROWS, _CW)
    x_pad = jnp.pad(x, ((0, _N_P - _N), (0, 0)))

    degp = _deg_kernel(dst_p).reshape(_TILES, _NROWS, _LANES)
    deg = _tc_deg_reduce(degp).reshape(_N_P, 1)
    g1 = _tc_layer1(x_pad, W1, deg)
    p = _agg_kernel(g1, src_p, dst_p)
    g2 = _tc_layer2(p, g1, deg, W2, b1.reshape(1, _D))
    q = _agg_kernel(g2, src_p, dst_p)
    return _tc_layer3(q, g2, deg, b2.reshape(1, _D))


# speedup vs baseline: 1.0057x; 1.0057x over previous
import functools

import jax
import jax.numpy as jnp
from jax import lax
from jax.experimental import pallas as pl
from jax.experimental.pallas import tpu as pltpu
from jax.experimental.pallas import tpu_sc as plsc

_N = 10000
_D = 128
_N_P = 10240
_E = 320000
_E_P = 327680
_LANES = 128
_CW = 64
_CROWS = _E_P // _CW
_NC = 2
_NS = 16
_TILES = _NC * _NS
_CROWS_PT = _CROWS // _TILES
_NROWS = _N_P // _LANES
_ACC_PT = _N_P // _NS
_TRASH = _N

_mesh = plsc.VectorSubcoreMesh(core_axis_name="c", subcore_axis_name="s")


@functools.partial(
    pl.kernel,
    out_type=jax.ShapeDtypeStruct((_TILES * _N_P,), jnp.float32),
    mesh=_mesh,
    compiler_params=pltpu.CompilerParams(needs_layout_passes=False),
    scratch_types=[
        pltpu.VMEM((_CROWS_PT, _CW), jnp.int32),
        pltpu.VMEM((_N_P,), jnp.float32),
        pltpu.SemaphoreType.DMA,
    ],
)
def _deg_kernel(dst_hbm, out_hbm, dst_v, cnt_v, sem):
    cid = lax.axis_index("c")
    sid = lax.axis_index("s")
    tid = cid * _NS + sid
    zero16 = jnp.zeros((16,), jnp.float32)

    def _zcnt(i, _):
        cnt_v[pl.ds(i * 16, 16)] = zero16
        return 0
    lax.fori_loop(0, _N_P // 16, _zcnt, 0)

    pltpu.sync_copy(dst_hbm.at[pl.ds(tid * _CROWS_PT, _CROWS_PT)], dst_v)

    ones16 = jnp.ones((16,), jnp.float32)

    def _cnt(i, _):
        r = i // 4
        k = i - r * 4
        idx = dst_v[r, pl.ds(k * 16, 16)]
        plsc.addupdate_scatter(cnt_v, [idx], ones16)
        return 0
    lax.fori_loop(0, _CROWS_PT * 4, _cnt, 0)

    pltpu.sync_copy(cnt_v, out_hbm.at[pl.ds(tid * _N_P, _N_P)])


def _tc_deg_reduce(degp):
    def body(p_ref, o_ref):
        o_ref[...] = jnp.sum(p_ref[...], axis=0)

    return pl.pallas_call(
        body,
        out_shape=jax.ShapeDtypeStruct((_NROWS, _LANES), jnp.float32),
    )(degp)


@functools.partial(
    pl.kernel,
    out_type=jax.ShapeDtypeStruct((_NC, _N_P, _D), jnp.float32),
    mesh=_mesh,
    compiler_params=pltpu.CompilerParams(needs_layout_passes=False),
    scratch_types=[
        pltpu.VMEM((_CROWS_PT // 4, _CW), jnp.int32),
        pltpu.VMEM((_CROWS_PT // 4, _CW), jnp.int32),
        pltpu.VMEM((_CW, _D), jnp.float32),
        pltpu.VMEM((_CW, _D), jnp.float32),
        pltpu.VMEM((_CW, _D), jnp.float32),
        pltpu.VMEM((_CW, _D), jnp.float32),
        pltpu.VMEM_SHARED((_N_P, _D), jnp.float32),
        pltpu.SemaphoreType.DMA,
        pltpu.SemaphoreType.DMA,
        pltpu.SemaphoreType.DMA,
        pltpu.SemaphoreType.DMA,
        pltpu.SemaphoreType.DMA,
        pltpu.SemaphoreType.DMA,
        pltpu.SemaphoreType.DMA,
        pltpu.SemaphoreType.DMA,
    ],
)
def _agg_kernel(g_hbm, src_hbm, dst_hbm, out_hbm,
                src_v, dst_v, buf0, buf1, buf2, buf3, acc_sh,
                gs0, gs1, gs2, gs3, ss0, ss1, ss2, ss3):
    cid = lax.axis_index("c")
    sid = lax.axis_index("s")
    tid = cid * _NS + sid
    bufs = [buf0, buf1, buf2, buf3]
    gs = [gs0, gs1, gs2, gs3]
    ss = [ss0, ss1, ss2, ss3]
    zero16 = jnp.zeros((16,), jnp.float32)

    def _z(i, _):
        r = i // 8
        k = i - r * 8
        buf0[r, pl.ds(k * 16, 16)] = zero16
        return 0
    lax.fori_loop(0, _CW * 8, _z, 0)

    def _zs(b, _):
        pltpu.sync_copy(buf0, acc_sh.at[pl.ds(sid * _ACC_PT + b * _CW, _CW)])
        return 0
    lax.fori_loop(0, _ACC_PT // _CW, _zs, 0)
    plsc.subcore_barrier()

    def _gather(c, b):
        pltpu.async_copy(g_hbm.at[src_v.at[c]], bufs[b], gs[b])

    def _gwait(b):
        pltpu.make_async_copy(g_hbm.at[src_v.at[0]], bufs[b], gs[b]).wait()

    def _scat(c, b):
        pltpu.async_copy(bufs[b], acc_sh.at[dst_v.at[c]], ss[b], add=True)

    def _swait(b):
        pltpu.make_async_copy(bufs[b], acc_sh.at[dst_v.at[0]], ss[b]).wait()

    nh = _CROWS_PT // 4
    nj = nh // 4
    for h in range(4):
        base = tid * _CROWS_PT + h * nh
        pltpu.sync_copy(src_hbm.at[pl.ds(base, nh)], src_v)
        pltpu.sync_copy(dst_hbm.at[pl.ds(base, nh)], dst_v)
        _gather(0, 0)
        _gather(1, 1)

        def _step(j, _):
            c0 = 4 * j
            _gwait(0)
            _scat(c0, 0)

            @pl.when(j > 0)
            def _():
                _swait(2)
            _gather(c0 + 2, 2)
            _gwait(1)
            _scat(c0 + 1, 1)

            @pl.when(j > 0)
            def _():
                _swait(3)
            _gather(c0 + 3, 3)
            _gwait(2)
            _scat(c0 + 2, 2)
            _swait(0)

            @pl.when(j < nj - 1)
            def _():
                _gather(c0 + 4, 0)
            _gwait(3)
            _scat(c0 + 3, 3)
            _swait(1)

            @pl.when(j < nj - 1)
            def _():
                _gather(c0 + 5, 1)
            return 0
        lax.fori_loop(0, nj, _step, 0)
        _swait(2)
        _swait(3)

    plsc.subcore_barrier()
    pltpu.sync_copy(acc_sh.at[pl.ds(sid * _ACC_PT, _ACC_PT)],
                    out_hbm.at[cid, pl.ds(sid * _ACC_PT, _ACC_PT)])


_BM = 1024
_BM3 = 1000


def _tc_layer1(x_pad, w1, deg):
    def body(x_ref, w_ref, d_ref, g_ref):
        dis = lax.rsqrt(d_ref[...] + 1.0)
        h = jnp.dot(x_ref[...], w_ref[...], preferred_element_type=jnp.float32)
        g_ref[...] = h * dis

    return pl.pallas_call(
        body,
        grid=(_N_P // _BM,),
        in_specs=[
            pl.BlockSpec((_BM, _D), lambda b: (b, 0)),
            pl.BlockSpec((_D, _D), lambda b: (0, 0)),
            pl.BlockSpec((_BM, 1), lambda b: (b, 0)),
        ],
        out_specs=pl.BlockSpec((_BM, _D), lambda b: (b, 0)),
        out_shape=jax.ShapeDtypeStruct((_N_P, _D), jnp.float32),
    )(x_pad, w1, deg)


def _tc_layer2(parts, g1, deg, w2, b1):
    def body(p_ref, g_ref, d_ref, w_ref, b_ref, o_ref):
        dis = lax.rsqrt(d_ref[...] + 1.0)
        s = p_ref[0] + p_ref[1] + g_ref[...]
        z = jnp.maximum(s * dis + b_ref[...], 0.0)
        o_ref[...] = jnp.dot(z, w_ref[...], preferred_element_type=jnp.float32) * dis

    return pl.pallas_call(
        body,
        grid=(_N_P // _BM,),
        in_specs=[
            pl.BlockSpec((_NC, _BM, _D), lambda b: (0, b, 0)),
            pl.BlockSpec((_BM, _D), lambda b: (b, 0)),
            pl.BlockSpec((_BM, 1), lambda b: (b, 0)),
            pl.BlockSpec((_D, _D), lambda b: (0, 0)),
            pl.BlockSpec((1, _D), lambda b: (0, 0)),
        ],
        out_specs=pl.BlockSpec((_BM, _D), lambda b: (b, 0)),
        out_shape=jax.ShapeDtypeStruct((_N_P, _D), jnp.float32),
    )(parts, g1, deg, w2, b1)


def _tc_layer3(parts, g2, deg, b2):
    def body(p_ref, g_ref, d_ref, b_ref, o_ref):
        dis = lax.rsqrt(d_ref[...] + 1.0)
        s = p_ref[0] + p_ref[1] + g_ref[...]
        o_ref[...] = s * dis + b_ref[...]

    return pl.pallas_call(
        body,
        grid=(_N // _BM3,),
        in_specs=[
            pl.BlockSpec((_NC, _BM3, _D), lambda b: (0, b, 0)),
            pl.BlockSpec((_BM3, _D), lambda b: (b, 0)),
            pl.BlockSpec((_BM3, 1), lambda b: (b, 0)),
            pl.BlockSpec((1, _D), lambda b: (0, 0)),
        ],
        out_specs=pl.BlockSpec((_BM3, _D), lambda b: (b, 0)),
        out_shape=jax.ShapeDtypeStruct((_N, _D), jnp.float32),
    )(parts, g2, deg, b2)


def kernel(x, edge_index, W1, b1, W2, b2):
    x = x.astype(jnp.float32)
    src = edge_index[0].astype(jnp.int32)
    dst = edge_index[1].astype(jnp.int32)
    src_p = jnp.concatenate(
        [src, jnp.zeros((_E_P - _E,), jnp.int32)]).reshape(_CROWS, _CW)
    pad_dst = _N + jnp.arange(_E_P - _E, dtype=jnp.int32) % (_N_P - _N)
    dst_p = jnp.concatenate([dst, pad_dst]).reshape(_CROWS, _CW)
    x_pad = jnp.pad(x, ((0, _N_P - _N), (0, 0)))

    degp = _deg_kernel(dst_p).reshape(_TILES, _NROWS, _LANES)
    deg = _tc_deg_reduce(degp).reshape(_N_P, 1)
    g1 = _tc_layer1(x_pad, W1, deg)
    p = _agg_kernel(g1, src_p, dst_p)
    g2 = _tc_layer2(p, g1, deg, W2, b1.reshape(1, _D))
    q = _agg_kernel(g2, src_p, dst_p)
    return _tc_layer3(q, g2, deg, b2.reshape(1, _D))


# speedup vs baseline: 3.0827x; 3.0652x over previous
import functools

import jax
import jax.numpy as jnp
from jax import lax
from jax.experimental import pallas as pl
from jax.experimental.pallas import tpu as pltpu
from jax.experimental.pallas import tpu_sc as plsc

_N = 10000
_D = 128
_N_P = 10240
_E = 320000
_E_P = 327680
_LANES = 128
_CW = 64
_CROWS = _E_P // _CW
_NC = 2
_NS = 16
_TILES = _NC * _NS
_CROWS_PT = _CROWS // _TILES
_NROWS = _N_P // _LANES
_ACC_PT = _N_P // _NS
_TRASH = _N

_mesh = plsc.VectorSubcoreMesh(core_axis_name="c", subcore_axis_name="s")


@functools.partial(
    pl.kernel,
    out_type=jax.ShapeDtypeStruct((_TILES * _N_P,), jnp.float32),
    mesh=_mesh,
    compiler_params=pltpu.CompilerParams(needs_layout_passes=False),
    scratch_types=[
        pltpu.VMEM((_CROWS_PT, _CW), jnp.int32),
        pltpu.VMEM((_N_P,), jnp.float32),
        pltpu.SemaphoreType.DMA,
    ],
)
def _deg_kernel(dst_hbm, out_hbm, dst_v, cnt_v, sem):
    cid = lax.axis_index("c")
    sid = lax.axis_index("s")
    tid = cid * _NS + sid
    zero16 = jnp.zeros((16,), jnp.float32)

    def _zcnt(i, _):
        cnt_v[pl.ds(i * 16, 16)] = zero16
        return 0
    lax.fori_loop(0, _N_P // 16, _zcnt, 0)

    pltpu.sync_copy(dst_hbm.at[pl.ds(tid * _CROWS_PT, _CROWS_PT)], dst_v)

    ones16 = jnp.ones((16,), jnp.float32)

    def _cnt(i, _):
        r = i // 4
        k = i - r * 4
        idx = dst_v[r, pl.ds(k * 16, 16)]
        plsc.addupdate_scatter(cnt_v, [idx], ones16)
        return 0
    lax.fori_loop(0, _CROWS_PT * 4, _cnt, 0)

    pltpu.sync_copy(cnt_v, out_hbm.at[pl.ds(tid * _N_P, _N_P)])


def _tc_deg_reduce(degp):
    def body(p_ref, o_ref):
        o_ref[...] = jnp.sum(p_ref[...], axis=0)

    return pl.pallas_call(
        body,
        out_shape=jax.ShapeDtypeStruct((_NROWS, _LANES), jnp.float32),
    )(degp)


@functools.partial(
    pl.kernel,
    out_type=jax.ShapeDtypeStruct((_NC, _N_P, _D), jnp.float32),
    mesh=_mesh,
    compiler_params=pltpu.CompilerParams(needs_layout_passes=False),
    scratch_types=[
        pltpu.VMEM((_CROWS_PT // 4, _CW), jnp.int32),
        pltpu.VMEM((_CROWS_PT // 4, _CW), jnp.int32),
        pltpu.VMEM((_CW, _D), jnp.float32),
        pltpu.VMEM((_CW, _D), jnp.float32),
        pltpu.VMEM((_CW, _D), jnp.float32),
        pltpu.VMEM((_CW, _D), jnp.float32),
        pltpu.VMEM_SHARED((_N_P, _D), jnp.float32),
        pltpu.SemaphoreType.DMA,
        pltpu.SemaphoreType.DMA,
        pltpu.SemaphoreType.DMA,
        pltpu.SemaphoreType.DMA,
        pltpu.SemaphoreType.DMA,
        pltpu.SemaphoreType.DMA,
        pltpu.SemaphoreType.DMA,
        pltpu.SemaphoreType.DMA,
    ],
)
def _agg_kernel(g_hbm, src_hbm, dst_hbm, out_hbm,
                src_v, dst_v, buf0, buf1, buf2, buf3, acc_sh,
                gs0, gs1, gs2, gs3, ss0, ss1, ss2, ss3):
    cid = lax.axis_index("c")
    sid = lax.axis_index("s")
    tid = cid * _NS + sid
    bufs = [buf0, buf1, buf2, buf3]
    gs = [gs0, gs1, gs2, gs3]
    ss = [ss0, ss1, ss2, ss3]
    zero16 = jnp.zeros((16,), jnp.float32)

    def _z(i, _):
        r = i // 8
        k = i - r * 8
        buf0[r, pl.ds(k * 16, 16)] = zero16
        return 0
    lax.fori_loop(0, _CW * 8, _z, 0)

    def _zs(b, _):
        pltpu.sync_copy(buf0, acc_sh.at[pl.ds(sid * _ACC_PT + b * _CW, _CW)])
        return 0
    lax.fori_loop(0, _ACC_PT // _CW, _zs, 0)
    plsc.subcore_barrier()

    def _gather(c, b):
        pltpu.async_copy(g_hbm.at[src_v.at[c]], bufs[b], gs[b])

    def _gwait(b):
        pltpu.make_async_copy(g_hbm.at[src_v.at[0]], bufs[b], gs[b]).wait()

    def _scat(c, b):
        pltpu.async_copy(bufs[b], acc_sh.at[dst_v.at[c]], ss[b], add=True)

    def _swait(b):
        pltpu.make_async_copy(bufs[b], acc_sh.at[dst_v.at[0]], ss[b]).wait()

    nh = _CROWS_PT // 4
    nj = nh // 4
    for h in range(4):
        base = tid * _CROWS_PT + h * nh
        pltpu.sync_copy(src_hbm.at[pl.ds(base, nh)], src_v)
        pltpu.sync_copy(dst_hbm.at[pl.ds(base, nh)], dst_v)
        _gather(0, 0)
        _gather(1, 1)

        def _step(j, _):
            c0 = 4 * j
            _gwait(0)
            _scat(c0, 0)

            @pl.when(j > 0)
            def _():
                _swait(2)
            _gather(c0 + 2, 2)
            _gwait(1)
            _scat(c0 + 1, 1)

            @pl.when(j > 0)
            def _():
                _swait(3)
            _gather(c0 + 3, 3)
            _gwait(2)
            _scat(c0 + 2, 2)
            _swait(0)

            @pl.when(j < nj - 1)
            def _():
                _gather(c0 + 4, 0)
            _gwait(3)
            _scat(c0 + 3, 3)
            _swait(1)

            @pl.when(j < nj - 1)
            def _():
                _gather(c0 + 5, 1)
            return 0
        lax.fori_loop(0, nj, _step, 0)
        _swait(2)
        _swait(3)

    plsc.subcore_barrier()
    pltpu.sync_copy(acc_sh.at[pl.ds(sid * _ACC_PT, _ACC_PT)],
                    out_hbm.at[cid, pl.ds(sid * _ACC_PT, _ACC_PT)])


_BM = 1024
_BM3 = 1000


def _tc_layer1(x_pad, w1, deg):
    def body(x_ref, w_ref, d_ref, g_ref):
        dis = lax.rsqrt(d_ref[...] + 1.0)
        h = jnp.dot(x_ref[...], w_ref[...], preferred_element_type=jnp.float32)
        g_ref[...] = h * dis

    return pl.pallas_call(
        body,
        grid=(_N_P // _BM,),
        in_specs=[
            pl.BlockSpec((_BM, _D), lambda b: (b, 0)),
            pl.BlockSpec((_D, _D), lambda b: (0, 0)),
            pl.BlockSpec((_BM, 1), lambda b: (b, 0)),
        ],
        out_specs=pl.BlockSpec((_BM, _D), lambda b: (b, 0)),
        out_shape=jax.ShapeDtypeStruct((_N_P, _D), jnp.float32),
    )(x_pad, w1, deg)


def _tc_layer2(parts, g1, deg, w2, b1):
    def body(p_ref, g_ref, d_ref, w_ref, b_ref, o_ref):
        dis = lax.rsqrt(d_ref[...] + 1.0)
        s = p_ref[0] + p_ref[1] + g_ref[...]
        z = jnp.maximum(s * dis + b_ref[...], 0.0)
        o_ref[...] = jnp.dot(z, w_ref[...], preferred_element_type=jnp.float32) * dis

    return pl.pallas_call(
        body,
        grid=(_N_P // _BM,),
        in_specs=[
            pl.BlockSpec((_NC, _BM, _D), lambda b: (0, b, 0)),
            pl.BlockSpec((_BM, _D), lambda b: (b, 0)),
            pl.BlockSpec((_BM, 1), lambda b: (b, 0)),
            pl.BlockSpec((_D, _D), lambda b: (0, 0)),
            pl.BlockSpec((1, _D), lambda b: (0, 0)),
        ],
        out_specs=pl.BlockSpec((_BM, _D), lambda b: (b, 0)),
        out_shape=jax.ShapeDtypeStruct((_N_P, _D), jnp.float32),
    )(parts, g1, deg, w2, b1)


def _tc_layer3(parts, g2, deg, b2):
    def body(p_ref, g_ref, d_ref, b_ref, o_ref):
        dis = lax.rsqrt(d_ref[...] + 1.0)
        s = p_ref[0] + p_ref[1] + g_ref[...]
        o_ref[...] = s * dis + b_ref[...]

    return pl.pallas_call(
        body,
        grid=(_N // _BM3,),
        in_specs=[
            pl.BlockSpec((_NC, _BM3, _D), lambda b: (0, b, 0)),
            pl.BlockSpec((_BM3, _D), lambda b: (b, 0)),
            pl.BlockSpec((_BM3, 1), lambda b: (b, 0)),
            pl.BlockSpec((1, _D), lambda b: (0, 0)),
        ],
        out_specs=pl.BlockSpec((_BM3, _D), lambda b: (b, 0)),
        out_shape=jax.ShapeDtypeStruct((_N, _D), jnp.float32),
    )(parts, g2, deg, b2)


def kernel(x, edge_index, W1, b1, W2, b2):
    x = x.astype(jnp.float32)
    src = edge_index[0].astype(jnp.int32)
    dst = edge_index[1].astype(jnp.int32)
    pad_idx = jnp.arange(_E_P - _E, dtype=jnp.int32)
    src_p = jnp.concatenate([src, pad_idx % _N]).reshape(_CROWS, _CW)
    pad_dst = _N + pad_idx % (_N_P - _N)
    dst_p = jnp.concatenate([dst, pad_dst]).reshape(_CROWS, _CW)
    x_pad = jnp.pad(x, ((0, _N_P - _N), (0, 0)))

    degp = _deg_kernel(dst_p).reshape(_TILES, _NROWS, _LANES)
    deg = _tc_deg_reduce(degp).reshape(_N_P, 1)
    g1 = _tc_layer1(x_pad, W1, deg)
    p = _agg_kernel(g1, src_p, dst_p)
    g2 = _tc_layer2(p, g1, deg, W2, b1.reshape(1, _D))
    q = _agg_kernel(g2, src_p, dst_p)
    return _tc_layer3(q, g2, deg, b2.reshape(1, _D))
